# baseline (speedup 1.0000x reference)
import functools

import jax
import jax.numpy as jnp
from jax import lax
from jax.experimental import pallas as pl
from jax.experimental.pallas import tpu as pltpu
from jax.experimental.pallas import tpu_sc as plsc

N = 10000
D = 128
RELW = 32
NSUB = 16
NCORE = 2
NWORK = NCORE * NSUB
EPW = 10000
K = 80
NBATCH = EPW // K
ROWS_A = 624


def _sc_aggregate(x, src, dst, rel, relaug, zg, zr):
    mesh = plsc.VectorSubcoreMesh(core_axis_name="c", subcore_axis_name="s")

    @functools.partial(
        pl.kernel,
        mesh=mesh,
        compiler_params=pltpu.CompilerParams(use_tc_tiling_on_sc=False),
        out_type=[
            jax.ShapeDtypeStruct((NCORE * N, D), jnp.float32),
            jax.ShapeDtypeStruct((NCORE * N, RELW), jnp.float32),
        ],
        scratch_types=[
            pltpu.VMEM_SHARED((N, D), jnp.float32),
            pltpu.VMEM_SHARED((N, RELW), jnp.float32),
            [pltpu.VMEM((K,), jnp.int32)] * 3,
            [pltpu.VMEM((K,), jnp.int32)] * 3,
            pltpu.VMEM((K, D), jnp.float32),
            pltpu.VMEM((K, D), jnp.float32),
            pltpu.VMEM((K, RELW), jnp.float32),
            pltpu.VMEM((K, RELW), jnp.float32),
            [pltpu.SemaphoreType.DMA] * 8,
        ],
    )
    def k(x_hbm, src_hbm, dst_hbm, rel_hbm, relaug_hbm, zg_hbm, zr_hbm,
          gout_hbm, rout_hbm, g_sp, r_sp, idxa, idxb,
          xa, xb, ra, rb, sems):
        semia, semib, semxa, semxb, semra, semrb, semg, semr2 = sems
        c = lax.axis_index("c")
        s = lax.axis_index("s")
        wid = c * NSUB + s
        rbase = s * ROWS_A
        ebase = wid * EPW

        pltpu.sync_copy(zg_hbm.at[pl.ds(0, ROWS_A)],
                        g_sp.at[pl.ds(rbase, ROWS_A)])
        pltpu.sync_copy(zr_hbm.at[pl.ds(0, ROWS_A)],
                        r_sp.at[pl.ds(rbase, ROWS_A)])

        @pl.when(s == NSUB - 1)
        def _():
            tail = NSUB * ROWS_A
            pltpu.sync_copy(zg_hbm.at[pl.ds(0, N - tail)],
                            g_sp.at[pl.ds(tail, N - tail)])
            pltpu.sync_copy(zr_hbm.at[pl.ds(0, N - tail)],
                            r_sp.at[pl.ds(tail, N - tail)])

        plsc.subcore_barrier()

        def fire_idx(j, bufs, sem):
            off = ebase + j * K
            pltpu.async_copy(src_hbm.at[pl.ds(off, K)], bufs[0], sem)
            pltpu.async_copy(dst_hbm.at[pl.ds(off, K)], bufs[1], sem)
            pltpu.async_copy(rel_hbm.at[pl.ds(off, K)], bufs[2], sem)

        def drain_idx(bufs, sem):
            for b in bufs:
                pltpu.make_async_copy(src_hbm.at[pl.ds(0, K)], b,
                                      sem).wait()

        def fire_rows(bufs, xbuf, rbuf, semx, semr):
            pltpu.async_copy(x_hbm.at[bufs[0]], xbuf, semx)
            pltpu.async_copy(relaug_hbm.at[bufs[2]], rbuf, semr)

        def drain_rows(bufs, xbuf, rbuf, semx, semr):
            pltpu.make_async_copy(x_hbm.at[bufs[0]], xbuf, semx).wait()
            pltpu.make_async_copy(relaug_hbm.at[bufs[2]], rbuf,
                                  semr).wait()

        def scat(bufs, xbuf, rbuf):
            pltpu.async_copy(xbuf, g_sp.at[bufs[1]], semg, add=True)
            pltpu.async_copy(rbuf, r_sp.at[bufs[1]], semr2, add=True)
            pltpu.make_async_copy(xbuf, g_sp.at[bufs[1]], semg).wait()
            pltpu.make_async_copy(rbuf, r_sp.at[bufs[1]], semr2).wait()

        fire_idx(0, idxa, semia)
        drain_idx(idxa, semia)
        fire_rows(idxa, xa, ra, semxa, semra)
        fire_idx(1, idxb, semib)

        def body(t, carry):
            ja = 2 * t
            jb = 2 * t + 1
            drain_idx(idxb, semib)
            fire_rows(idxb, xb, rb, semxb, semrb)
            drain_rows(idxa, xa, ra, semxa, semra)
            scat(idxa, xa, ra)
            fire_idx(ja + 2, idxa, semia)
            drain_idx(idxa, semia)
            fire_rows(idxa, xa, ra, semxa, semra)
            drain_rows(idxb, xb, rb, semxb, semrb)
            scat(idxb, xb, rb)
            jnext = jnp.minimum(jb + 2, NBATCH - 1)
            fire_idx(jnext, idxb, semib)
            return carry

        lax.fori_loop(0, (NBATCH - 1) // 2, body, 0)
        drain_idx(idxb, semib)
        drain_rows(idxa, xa, ra, semxa, semra)
        scat(idxa, xa, ra)
        plsc.subcore_barrier()

        obase = c * N + rbase
        pltpu.sync_copy(g_sp.at[pl.ds(rbase, ROWS_A)],
                        gout_hbm.at[pl.ds(obase, ROWS_A)])
        pltpu.sync_copy(r_sp.at[pl.ds(rbase, ROWS_A)],
                        rout_hbm.at[pl.ds(obase, ROWS_A)])

        @pl.when(s == NSUB - 1)
        def _():
            tail = NSUB * ROWS_A
            pltpu.sync_copy(g_sp.at[pl.ds(tail, N - tail)],
                            gout_hbm.at[pl.ds(c * N + tail, N - tail)])
            pltpu.sync_copy(r_sp.at[pl.ds(tail, N - tail)],
                            rout_hbm.at[pl.ds(c * N + tail, N - tail)])

    return k(x, src, dst, rel, relaug, zg, zr)


def _dense(x, g0, g1, r0, r1, wselfT, wsrcT, wrelaug, wneighT, bias):
    BN = 2000

    def body(x_r, g0_r, g1_r, r0_r, r1_r, ws_r, wsrc_r, wrel_r, wn_r, b_r,
             o_r):
        G = g0_r[...] + g1_r[...]
        R = r0_r[...] + r1_r[...]
        pre = (jnp.dot(G, wsrc_r[...], preferred_element_type=jnp.float32)
               + jnp.dot(R, wrel_r[...], preferred_element_type=jnp.float32))
        deg = R[:, 16:17]
        agg = pre / jnp.maximum(deg, 1.0)
        out = (jnp.dot(x_r[...], ws_r[...], preferred_element_type=jnp.float32)
               + jnp.dot(agg, wn_r[...], preferred_element_type=jnp.float32)
               + b_r[...])
        o_r[...] = jnp.maximum(out, 0.0)

    full = pl.BlockSpec((D, D), lambda i: (0, 0))
    blk = pl.BlockSpec((BN, D), lambda i: (i, 0))
    blkr = pl.BlockSpec((BN, RELW), lambda i: (i, 0))
    return pl.pallas_call(
        body,
        grid=(N // BN,),
        in_specs=[blk, blk, blk, blkr, blkr, full, full,
                  pl.BlockSpec((RELW, D), lambda i: (0, 0)),
                  full,
                  pl.BlockSpec((1, D), lambda i: (0, 0))],
        out_specs=blk,
        out_shape=jax.ShapeDtypeStruct((N, D), jnp.float32),
    )(x, g0, g1, r0, r1, wselfT, wsrcT, wrelaug, wneighT, bias)


def kernel(x, edge_src, edge_dst, rel_ids, rel_emb, W_msg, b_msg,
           W_self, b_self, W_neigh, b_neigh):
    src = edge_src.astype(jnp.int32)
    dst = edge_dst.astype(jnp.int32)
    rel = rel_ids.astype(jnp.int32)
    nrel = rel_emb.shape[0]
    relaug = jnp.concatenate(
        [rel_emb.astype(jnp.float32),
         jnp.ones((nrel, 1), jnp.float32),
         jnp.zeros((nrel, RELW - rel_emb.shape[1] - 1), jnp.float32)],
        axis=1)
    zg = jnp.zeros((640, D), jnp.float32)
    zr = jnp.zeros((640, RELW), jnp.float32)

    gout, rout = _sc_aggregate(x, src, dst, rel, relaug, zg, zr)
    g0, g1 = gout[:N], gout[N:]
    r0, r1 = rout[:N], rout[N:]

    wsrcT = W_msg[:, :D].T
    wrelaug = jnp.concatenate(
        [W_msg[:, D:].T, b_msg[None, :],
         jnp.zeros((RELW - rel_emb.shape[1] - 1, D), jnp.float32)],
        axis=0)
    bias = (b_self + b_neigh)[None, :]
    return _dense(x, g0, g1, r0, r1, W_self.T, wsrcT, wrelaug,
                  W_neigh.T, bias)


# speedup vs baseline: 1.0124x; 1.0124x over previous
import functools

import jax
import jax.numpy as jnp
from jax import lax
from jax.experimental import pallas as pl
from jax.experimental.pallas import tpu as pltpu
from jax.experimental.pallas import tpu_sc as plsc

N = 10000
D = 128
RELW = 32
NSUB = 16
NCORE = 2
NWORK = NCORE * NSUB
EPW = 10000
K = 80
NBATCH = EPW // K
ROWS_A = 624


def _sc_aggregate(x, src, dst, rel, relaug, zg, zr):
    mesh = plsc.VectorSubcoreMesh(core_axis_name="c", subcore_axis_name="s")

    @functools.partial(
        pl.kernel,
        mesh=mesh,
        compiler_params=pltpu.CompilerParams(use_tc_tiling_on_sc=False),
        out_type=[
            jax.ShapeDtypeStruct((NCORE * N, D), jnp.float32),
            jax.ShapeDtypeStruct((NCORE * N, RELW), jnp.float32),
        ],
        scratch_types=[
            pltpu.VMEM_SHARED((N, D), jnp.float32),
            pltpu.VMEM_SHARED((N, RELW), jnp.float32),
            [pltpu.VMEM((K,), jnp.int32)] * 3,
            [pltpu.VMEM((K,), jnp.int32)] * 3,
            [pltpu.VMEM((K,), jnp.int32)] * 3,
            [pltpu.VMEM((K,), jnp.int32)] * 3,
            pltpu.VMEM((K, D), jnp.float32),
            pltpu.VMEM((K, D), jnp.float32),
            pltpu.VMEM((K, RELW), jnp.float32),
            pltpu.VMEM((K, RELW), jnp.float32),
            [pltpu.SemaphoreType.DMA] * 10,
        ],
    )
    def k(x_hbm, src_hbm, dst_hbm, rel_hbm, relaug_hbm, zg_hbm, zr_hbm,
          gout_hbm, rout_hbm, g_sp, r_sp, iA, iB, iC, iD,
          xa, xb, ra, rb, sems):
        (semA, semB, semC, semD, semxa, semxb, semra, semrb,
         semg, semr2) = sems
        c = lax.axis_index("c")
        s = lax.axis_index("s")
        wid = c * NSUB + s
        rbase = s * ROWS_A
        ebase = wid * EPW

        def fire_idx(j, bufs, sem):
            off = ebase + j * K
            pltpu.async_copy(src_hbm.at[pl.ds(off, K)], bufs[0], sem)
            pltpu.async_copy(dst_hbm.at[pl.ds(off, K)], bufs[1], sem)
            pltpu.async_copy(rel_hbm.at[pl.ds(off, K)], bufs[2], sem)

        def drain_idx(bufs, sem):
            for b in bufs:
                pltpu.make_async_copy(src_hbm.at[pl.ds(0, K)], b,
                                      sem).wait()

        def fire_rows(bufs, xbuf, rbuf, semx, semr):
            pltpu.async_copy(x_hbm.at[bufs[0]], xbuf, semx)
            pltpu.async_copy(relaug_hbm.at[bufs[2]], rbuf, semr)

        def drain_rows(bufs, xbuf, rbuf, semx, semr):
            pltpu.make_async_copy(x_hbm.at[bufs[0]], xbuf, semx).wait()
            pltpu.make_async_copy(relaug_hbm.at[bufs[2]], rbuf,
                                  semr).wait()

        def scat(bufs, xbuf, rbuf):
            pltpu.async_copy(xbuf, g_sp.at[bufs[1]], semg, add=True)
            pltpu.async_copy(rbuf, r_sp.at[bufs[1]], semr2, add=True)
            pltpu.make_async_copy(xbuf, g_sp.at[bufs[1]], semg).wait()
            pltpu.make_async_copy(rbuf, r_sp.at[bufs[1]], semr2).wait()

        fire_idx(0, iA, semA)
        fire_idx(1, iB, semB)
        fire_idx(2, iC, semC)
        fire_idx(3, iD, semD)

        pltpu.sync_copy(zg_hbm.at[pl.ds(0, ROWS_A)],
                        g_sp.at[pl.ds(rbase, ROWS_A)])
        pltpu.sync_copy(zr_hbm.at[pl.ds(0, ROWS_A)],
                        r_sp.at[pl.ds(rbase, ROWS_A)])

        @pl.when(s == NSUB - 1)
        def _():
            tail = NSUB * ROWS_A
            pltpu.sync_copy(zg_hbm.at[pl.ds(0, N - tail)],
                            g_sp.at[pl.ds(tail, N - tail)])
            pltpu.sync_copy(zr_hbm.at[pl.ds(0, N - tail)],
                            r_sp.at[pl.ds(tail, N - tail)])

        drain_idx(iA, semA)
        fire_rows(iA, xa, ra, semxa, semra)
        plsc.subcore_barrier()

        def body(u, carry):
            t0 = 4 * u
            drain_idx(iB, semB)
            fire_rows(iB, xb, rb, semxb, semrb)
            drain_rows(iA, xa, ra, semxa, semra)
            scat(iA, xa, ra)
            fire_idx(jnp.minimum(t0 + 4, NBATCH - 1), iA, semA)
            drain_idx(iC, semC)
            fire_rows(iC, xa, ra, semxa, semra)
            drain_rows(iB, xb, rb, semxb, semrb)
            scat(iB, xb, rb)
            fire_idx(jnp.minimum(t0 + 5, NBATCH - 1), iB, semB)
            drain_idx(iD, semD)
            fire_rows(iD, xb, rb, semxb, semrb)
            drain_rows(iC, xa, ra, semxa, semra)
            scat(iC, xa, ra)
            fire_idx(jnp.minimum(t0 + 6, NBATCH - 1), iC, semC)
            drain_idx(iA, semA)
            fire_rows(iA, xa, ra, semxa, semra)
            drain_rows(iD, xb, rb, semxb, semrb)
            scat(iD, xb, rb)
            fire_idx(jnp.minimum(t0 + 7, NBATCH - 1), iD, semD)
            return carry

        lax.fori_loop(0, (NBATCH - 1) // 4, body, 0)
        drain_idx(iB, semB)
        drain_idx(iC, semC)
        drain_idx(iD, semD)
        drain_rows(iA, xa, ra, semxa, semra)
        scat(iA, xa, ra)
        plsc.subcore_barrier()

        obase = c * N + rbase
        pltpu.sync_copy(g_sp.at[pl.ds(rbase, ROWS_A)],
                        gout_hbm.at[pl.ds(obase, ROWS_A)])
        pltpu.sync_copy(r_sp.at[pl.ds(rbase, ROWS_A)],
                        rout_hbm.at[pl.ds(obase, ROWS_A)])

        @pl.when(s == NSUB - 1)
        def _():
            tail = NSUB * ROWS_A
            pltpu.sync_copy(g_sp.at[pl.ds(tail, N - tail)],
                            gout_hbm.at[pl.ds(c * N + tail, N - tail)])
            pltpu.sync_copy(r_sp.at[pl.ds(tail, N - tail)],
                            rout_hbm.at[pl.ds(c * N + tail, N - tail)])

    return k(x, src, dst, rel, relaug, zg, zr)


def _dense(x, g0, g1, r0, r1, wselfT, wsrcT, wrelaug, wneighT, bias):
    BN = 2000

    def body(x_r, g0_r, g1_r, r0_r, r1_r, ws_r, wsrc_r, wrel_r, wn_r, b_r,
             o_r):
        G = g0_r[...] + g1_r[...]
        R = r0_r[...] + r1_r[...]
        pre = (jnp.dot(G, wsrc_r[...], preferred_element_type=jnp.float32)
               + jnp.dot(R, wrel_r[...], preferred_element_type=jnp.float32))
        deg = R[:, 16:17]
        agg = pre / jnp.maximum(deg, 1.0)
        out = (jnp.dot(x_r[...], ws_r[...], preferred_element_type=jnp.float32)
               + jnp.dot(agg, wn_r[...], preferred_element_type=jnp.float32)
               + b_r[...])
        o_r[...] = jnp.maximum(out, 0.0)

    full = pl.BlockSpec((D, D), lambda i: (0, 0))
    blk = pl.BlockSpec((BN, D), lambda i: (i, 0))
    blkr = pl.BlockSpec((BN, RELW), lambda i: (i, 0))
    return pl.pallas_call(
        body,
        grid=(N // BN,),
        in_specs=[blk, blk, blk, blkr, blkr, full, full,
                  pl.BlockSpec((RELW, D), lambda i: (0, 0)),
                  full,
                  pl.BlockSpec((1, D), lambda i: (0, 0))],
        out_specs=blk,
        out_shape=jax.ShapeDtypeStruct((N, D), jnp.float32),
    )(x, g0, g1, r0, r1, wselfT, wsrcT, wrelaug, wneighT, bias)


def kernel(x, edge_src, edge_dst, rel_ids, rel_emb, W_msg, b_msg,
           W_self, b_self, W_neigh, b_neigh):
    src = edge_src.astype(jnp.int32)
    dst = edge_dst.astype(jnp.int32)
    rel = rel_ids.astype(jnp.int32)
    nrel = rel_emb.shape[0]
    relaug = jnp.concatenate(
        [rel_emb.astype(jnp.float32),
         jnp.ones((nrel, 1), jnp.float32),
         jnp.zeros((nrel, RELW - rel_emb.shape[1] - 1), jnp.float32)],
        axis=1)
    zg = jnp.zeros((640, D), jnp.float32)
    zr = jnp.zeros((640, RELW), jnp.float32)

    gout, rout = _sc_aggregate(x, src, dst, rel, relaug, zg, zr)
    g0, g1 = gout[:N], gout[N:]
    r0, r1 = rout[:N], rout[N:]

    wsrcT = W_msg[:, :D].T
    wrelaug = jnp.concatenate(
        [W_msg[:, D:].T, b_msg[None, :],
         jnp.zeros((RELW - rel_emb.shape[1] - 1, D), jnp.float32)],
        axis=0)
    bias = (b_self + b_neigh)[None, :]
    return _dense(x, g0, g1, r0, r1, W_self.T, wsrcT, wrelaug,
                  W_neigh.T, bias)


# speedup vs baseline: 1.0191x; 1.0067x over previous
import functools

import jax
import jax.numpy as jnp
from jax import lax
from jax.experimental import pallas as pl
from jax.experimental.pallas import tpu as pltpu
from jax.experimental.pallas import tpu_sc as plsc

N = 10000
D = 128
RELW = 32
NSUB = 16
NCORE = 2
NWORK = NCORE * NSUB
EPW = 10000
K = 80
NBATCH = EPW // K
ROWS_A = 624


def _sc_aggregate(x, src, dst, rel, relaug, zg, zr):
    mesh = plsc.VectorSubcoreMesh(core_axis_name="c", subcore_axis_name="s")

    @functools.partial(
        pl.kernel,
        mesh=mesh,
        compiler_params=pltpu.CompilerParams(use_tc_tiling_on_sc=False),
        out_type=[
            jax.ShapeDtypeStruct((NCORE * N, D), jnp.float32),
            jax.ShapeDtypeStruct((NCORE * N, RELW), jnp.float32),
        ],
        scratch_types=[
            pltpu.VMEM_SHARED((N, D), jnp.float32),
            pltpu.VMEM_SHARED((N, RELW), jnp.float32),
            [pltpu.VMEM((K,), jnp.int32)] * 3,
            [pltpu.VMEM((K,), jnp.int32)] * 3,
            [pltpu.VMEM((K,), jnp.int32)] * 3,
            [pltpu.VMEM((K,), jnp.int32)] * 3,
            pltpu.VMEM((K, D), jnp.float32),
            pltpu.VMEM((K, D), jnp.float32),
            pltpu.VMEM((K, RELW), jnp.float32),
            pltpu.VMEM((K, RELW), jnp.float32),
            [pltpu.SemaphoreType.DMA] * 10,
        ],
    )
    def k(x_hbm, src_hbm, dst_hbm, rel_hbm, relaug_hbm, zg_hbm, zr_hbm,
          gout_hbm, rout_hbm, g_sp, r_sp, iA, iB, iC, iD,
          xa, xb, ra, rb, sems):
        (semA, semB, semC, semD, semxa, semxb, semra, semrb,
         semg, semr2) = sems
        c = lax.axis_index("c")
        s = lax.axis_index("s")
        wid = c * NSUB + s
        rbase = s * ROWS_A
        ebase = wid * EPW

        def fire_idx(j, bufs, sem):
            off = ebase + j * K
            pltpu.async_copy(src_hbm.at[pl.ds(off, K)], bufs[0], sem)
            pltpu.async_copy(dst_hbm.at[pl.ds(off, K)], bufs[1], sem)
            pltpu.async_copy(rel_hbm.at[pl.ds(off, K)], bufs[2], sem)

        def drain_idx(bufs, sem):
            for b in bufs:
                pltpu.make_async_copy(src_hbm.at[pl.ds(0, K)], b,
                                      sem).wait()

        def fire_rows(bufs, xbuf, rbuf, semx, semr):
            pltpu.async_copy(x_hbm.at[bufs[0]], xbuf, semx)
            pltpu.async_copy(relaug_hbm.at[bufs[2]], rbuf, semr)

        def drain_rows(bufs, xbuf, rbuf, semx, semr):
            pltpu.make_async_copy(x_hbm.at[bufs[0]], xbuf, semx).wait()
            pltpu.make_async_copy(relaug_hbm.at[bufs[2]], rbuf,
                                  semr).wait()

        def scat(bufs, xbuf, rbuf):
            pltpu.async_copy(xbuf, g_sp.at[bufs[1]], semg, add=True)
            pltpu.async_copy(rbuf, r_sp.at[bufs[1]], semr2, add=True)
            pltpu.make_async_copy(xbuf, g_sp.at[bufs[1]], semg).wait()
            pltpu.make_async_copy(rbuf, r_sp.at[bufs[1]], semr2).wait()

        fire_idx(0, iA, semA)
        fire_idx(1, iB, semB)
        fire_idx(2, iC, semC)
        fire_idx(3, iD, semD)

        pltpu.sync_copy(zg_hbm.at[pl.ds(0, ROWS_A)],
                        g_sp.at[pl.ds(rbase, ROWS_A)])
        pltpu.sync_copy(zr_hbm.at[pl.ds(0, ROWS_A)],
                        r_sp.at[pl.ds(rbase, ROWS_A)])

        @pl.when(s == NSUB - 1)
        def _():
            tail = NSUB * ROWS_A
            pltpu.sync_copy(zg_hbm.at[pl.ds(0, N - tail)],
                            g_sp.at[pl.ds(tail, N - tail)])
            pltpu.sync_copy(zr_hbm.at[pl.ds(0, N - tail)],
                            r_sp.at[pl.ds(tail, N - tail)])

        drain_idx(iA, semA)
        fire_rows(iA, xa, ra, semxa, semra)
        plsc.subcore_barrier()

        def body(u, carry):
            t0 = 4 * u
            drain_idx(iB, semB)
            fire_rows(iB, xb, rb, semxb, semrb)
            drain_rows(iA, xa, ra, semxa, semra)
            scat(iA, xa, ra)
            fire_idx(jnp.minimum(t0 + 4, NBATCH - 1), iA, semA)
            drain_idx(iC, semC)
            fire_rows(iC, xa, ra, semxa, semra)
            drain_rows(iB, xb, rb, semxb, semrb)
            scat(iB, xb, rb)
            fire_idx(jnp.minimum(t0 + 5, NBATCH - 1), iB, semB)
            drain_idx(iD, semD)
            fire_rows(iD, xb, rb, semxb, semrb)
            drain_rows(iC, xa, ra, semxa, semra)
            scat(iC, xa, ra)
            fire_idx(jnp.minimum(t0 + 6, NBATCH - 1), iC, semC)
            drain_idx(iA, semA)
            fire_rows(iA, xa, ra, semxa, semra)
            drain_rows(iD, xb, rb, semxb, semrb)
            scat(iD, xb, rb)
            fire_idx(jnp.minimum(t0 + 7, NBATCH - 1), iD, semD)
            return carry

        lax.fori_loop(0, (NBATCH - 1) // 4, body, 0)
        drain_idx(iB, semB)
        drain_idx(iC, semC)
        drain_idx(iD, semD)
        drain_rows(iA, xa, ra, semxa, semra)
        scat(iA, xa, ra)
        plsc.subcore_barrier()

        obase = c * N + rbase
        pltpu.sync_copy(g_sp.at[pl.ds(rbase, ROWS_A)],
                        gout_hbm.at[pl.ds(obase, ROWS_A)])
        pltpu.sync_copy(r_sp.at[pl.ds(rbase, ROWS_A)],
                        rout_hbm.at[pl.ds(obase, ROWS_A)])

        @pl.when(s == NSUB - 1)
        def _():
            tail = NSUB * ROWS_A
            pltpu.sync_copy(g_sp.at[pl.ds(tail, N - tail)],
                            gout_hbm.at[pl.ds(c * N + tail, N - tail)])
            pltpu.sync_copy(r_sp.at[pl.ds(tail, N - tail)],
                            rout_hbm.at[pl.ds(c * N + tail, N - tail)])

    return k(x, src, dst, rel, relaug, zg, zr)


def _dense(x, gout, rout, W_msg, b_msg, W_self, b_self, W_neigh, b_neigh):
    BN = 2000
    NB = N // BN
    MSG_IN = W_msg.shape[1]

    def dotT(a, w):
        return lax.dot_general(a, w, (((1,), (1,)), ((), ())),
                               preferred_element_type=jnp.float32)

    def body(x_r, g0_r, g1_r, r0_r, r1_r, wm_r, bm_r, ws_r, bs_r, wn_r,
             bn_r, o_r):
        G = g0_r[...] + g1_r[...]
        R = r0_r[...] + r1_r[...]
        deg = R[:, 16:17]
        wm = wm_r[...]
        pre = (dotT(G, wm[:, :D]) + dotT(R[:, :16], wm[:, D:])
               + deg * bm_r[...])
        agg = pre / jnp.maximum(deg, 1.0)
        out = (dotT(x_r[...], ws_r[...]) + dotT(agg, wn_r[...])
               + bs_r[...] + bn_r[...])
        o_r[...] = jnp.maximum(out, 0.0)

    full = pl.BlockSpec((D, D), lambda i: (0, 0))
    blk = pl.BlockSpec((BN, D), lambda i: (i, 0))
    blkr = pl.BlockSpec((BN, RELW), lambda i: (i, 0))
    vec = pl.BlockSpec((1, D), lambda i: (0, 0))
    return pl.pallas_call(
        body,
        grid=(NB,),
        in_specs=[blk,
                  blk, pl.BlockSpec((BN, D), lambda i: (NB + i, 0)),
                  blkr, pl.BlockSpec((BN, RELW), lambda i: (NB + i, 0)),
                  pl.BlockSpec((D, MSG_IN), lambda i: (0, 0)),
                  vec, full, vec, full, vec],
        out_specs=blk,
        out_shape=jax.ShapeDtypeStruct((N, D), jnp.float32),
    )(x, gout, gout, rout, rout, W_msg, b_msg.reshape(1, D),
      W_self, b_self.reshape(1, D), W_neigh, b_neigh.reshape(1, D))


def kernel(x, edge_src, edge_dst, rel_ids, rel_emb, W_msg, b_msg,
           W_self, b_self, W_neigh, b_neigh):
    src = edge_src.astype(jnp.int32)
    dst = edge_dst.astype(jnp.int32)
    rel = rel_ids.astype(jnp.int32)
    nrel = rel_emb.shape[0]
    relaug = jnp.concatenate(
        [rel_emb.astype(jnp.float32),
         jnp.ones((nrel, 1), jnp.float32),
         jnp.zeros((nrel, RELW - rel_emb.shape[1] - 1), jnp.float32)],
        axis=1)
    zg = jnp.zeros((640, D), jnp.float32)
    zr = jnp.zeros((640, RELW), jnp.float32)

    gout, rout = _sc_aggregate(x, src, dst, rel, relaug, zg, zr)
    return _dense(x, gout, rout, W_msg, b_msg, W_self, b_self,
                  W_neigh, b_neigh)
